# Initial kernel scaffold; baseline (speedup 1.0000x reference)
import jax
import jax.numpy as jnp
from jax.experimental import pallas as pl


def kernel(x, table):
    raise NotImplementedError("write your pallas kernel here")


# speedup vs baseline: 1.5469x; 1.5469x over previous
import functools

import jax
import jax.numpy as jnp
from jax import lax
from jax.experimental import pallas as pl
from jax.experimental.pallas import tpu as pltpu
from jax.experimental.pallas import tpu_sc as plsc


def _make_gather(N, V, D, NC, NS):
    NW = NC * NS
    n_per_w = N // NW
    C = 1024
    n_chunks = n_per_w // C

    mesh = plsc.VectorSubcoreMesh(core_axis_name="c", subcore_axis_name="s")

    @functools.partial(
        pl.kernel,
        mesh=mesh,
        out_type=jax.ShapeDtypeStruct((N, D), jnp.float32),
        scratch_types=[
            pltpu.VMEM((C,), jnp.int32),
            pltpu.VMEM((C, D), jnp.float32),
            pltpu.SemaphoreType.DMA,
        ],
        compiler_params=pltpu.CompilerParams(use_tc_tiling_on_sc=False),
    )
    def gather_kernel(idx_hbm, table_hbm, out_hbm, idx_v, rows_v, sem):
        wid = lax.axis_index("s") * NC + lax.axis_index("c")
        base = wid * n_per_w

        def body(i, carry):
            off = pl.multiple_of(base + i * C, 8)
            pltpu.sync_copy(idx_hbm.at[pl.ds(off, C)], idx_v)
            pltpu.async_copy(table_hbm.at[idx_v], rows_v, sem).wait()
            pltpu.sync_copy(rows_v, out_hbm.at[pl.ds(off, C)])
            return carry

        lax.fori_loop(0, n_chunks, body, 0)

    return gather_kernel


def kernel(x, table):
    B, F = x.shape
    V, D = table.shape
    N = B * F
    info = plsc.get_sparse_core_info()
    gather = _make_gather(N, V, D, info.num_cores, info.num_subcores)
    flat = gather(x.reshape(-1).astype(jnp.int32), table)
    return flat.reshape(B, F, D)


# speedup vs baseline: 1.5778x; 1.0200x over previous
import functools

import jax
import jax.numpy as jnp
from jax import lax
from jax.experimental import pallas as pl
from jax.experimental.pallas import tpu as pltpu
from jax.experimental.pallas import tpu_sc as plsc


def _make_gather(N, V, D, NC, NS):
    NW = NC * NS
    n_per_w = N // NW
    C = 832
    n_chunks = n_per_w // C
    n_pairs = n_chunks // 2

    mesh = plsc.VectorSubcoreMesh(core_axis_name="c", subcore_axis_name="s")

    @functools.partial(
        pl.kernel,
        mesh=mesh,
        out_type=jax.ShapeDtypeStruct((N, D), jnp.float32),
        scratch_types=[
            pltpu.VMEM((n_per_w,), jnp.int32),
            pltpu.VMEM((C, D), jnp.float32),
            pltpu.VMEM((C, D), jnp.float32),
            pltpu.SemaphoreType.DMA,
            pltpu.SemaphoreType.DMA,
        ],
        compiler_params=pltpu.CompilerParams(use_tc_tiling_on_sc=False),
    )
    def gather_kernel(idx_hbm, table_hbm, out_hbm, idx_v, rows0, rows1, sem0, sem1):
        wid = lax.axis_index("s") * NC + lax.axis_index("c")
        base = pl.multiple_of(wid * n_per_w, 8)

        pltpu.sync_copy(idx_hbm.at[pl.ds(base, n_per_w)], idx_v)

        def gather_start(i, rows, sem):
            off = pl.multiple_of(i * C, 8)
            pltpu.async_copy(table_hbm.at[idx_v.at[pl.ds(off, C)]], rows, sem)

        def gather_wait(rows, sem):
            pltpu.make_async_copy(
                table_hbm.at[idx_v.at[pl.ds(0, C)]], rows, sem
            ).wait()

        def store(i, rows):
            off = pl.multiple_of(base + i * C, 8)
            pltpu.sync_copy(rows, out_hbm.at[pl.ds(off, C)])

        gather_start(0, rows0, sem0)

        def body(j, carry):
            i0 = 2 * j
            gather_start(i0 + 1, rows1, sem1)
            gather_wait(rows0, sem0)
            store(i0, rows0)

            @pl.when(j + 1 < n_pairs)
            def _():
                gather_start(i0 + 2, rows0, sem0)

            gather_wait(rows1, sem1)
            store(i0 + 1, rows1)
            return carry

        lax.fori_loop(0, n_pairs, body, 0)

    return gather_kernel


def kernel(x, table):
    B, F = x.shape
    V, D = table.shape
    N = B * F
    info = plsc.get_sparse_core_info()
    gather = _make_gather(N, V, D, info.num_cores, info.num_subcores)
    flat = gather(x.reshape(-1).astype(jnp.int32), table)
    return flat.reshape(B, F, D)


# speedup vs baseline: 1.5780x; 1.0001x over previous
import functools

import jax
import jax.numpy as jnp
from jax import lax
from jax.experimental import pallas as pl
from jax.experimental.pallas import tpu as pltpu
from jax.experimental.pallas import tpu_sc as plsc


def _make_gather(N, V, D, NC, NS):
    NW = NC * NS
    n_per_w = N // NW
    C = 832
    K = 4
    n_chunks = n_per_w // C
    n_rounds = n_chunks // K

    mesh = plsc.VectorSubcoreMesh(core_axis_name="c", subcore_axis_name="s")

    @functools.partial(
        pl.kernel,
        mesh=mesh,
        out_type=jax.ShapeDtypeStruct((N, D), jnp.float32),
        scratch_types=[
            pltpu.VMEM((n_per_w,), jnp.int32),
            [pltpu.VMEM((C, D), jnp.float32) for _ in range(K)],
            [pltpu.SemaphoreType.DMA for _ in range(K)],
        ],
        compiler_params=pltpu.CompilerParams(use_tc_tiling_on_sc=False),
    )
    def gather_kernel(idx_hbm, table_hbm, out_hbm, idx_v, rows, sems):
        wid = lax.axis_index("s") * NC + lax.axis_index("c")
        base = pl.multiple_of(wid * n_per_w, 8)

        pltpu.sync_copy(idx_hbm.at[pl.ds(base, n_per_w)], idx_v)

        def gather_start(i, b):
            off = pl.multiple_of(i * C, 8)
            pltpu.async_copy(table_hbm.at[idx_v.at[pl.ds(off, C)]], rows[b], sems[b])

        def gather_wait(b):
            pltpu.make_async_copy(
                table_hbm.at[idx_v.at[pl.ds(0, C)]], rows[b], sems[b]
            ).wait()

        def store(i, b):
            off = pl.multiple_of(base + i * C, 8)
            pltpu.sync_copy(rows[b], out_hbm.at[pl.ds(off, C)])

        for b in range(K):
            gather_start(b, b)

        def body(j, carry):
            for b in range(K):
                i = j * K + b
                gather_wait(b)
                store(i, b)

                @pl.when(i + K < n_chunks)
                def _():
                    gather_start(i + K, b)

            return carry

        lax.fori_loop(0, n_rounds, body, 0)

    return gather_kernel


def kernel(x, table):
    B, F = x.shape
    V, D = table.shape
    N = B * F
    info = plsc.get_sparse_core_info()
    gather = _make_gather(N, V, D, info.num_cores, info.num_subcores)
    flat = gather(x.reshape(-1).astype(jnp.int32), table)
    return flat.reshape(B, F, D)
